# baseline (speedup 1.0000x reference)
import functools

import jax
import jax.numpy as jnp
from jax import lax
from jax.experimental import pallas as pl
from jax.experimental.pallas import tpu as pltpu
from jax.experimental.pallas import tpu_sc as plsc

_B = 1024
_V = 100000
_H = 200
_L = 16

_NC = 2
_NS = 16
_NW = _NC * _NS
_TOTAL = _B * _H
_XN = _B * _V
_PER_W = _TOTAL // _NW
_CHUNK = 128
_STREAMS = _PER_W // _CHUNK
_FIRE = 10


def _gather_body(x1d_hbm, idx_hbm, out_hbm, idx_v, out_v, sem):
    wid = lax.axis_index("s") * _NC + lax.axis_index("c")
    gbase = wid * _PER_W
    pltpu.sync_copy(idx_hbm.at[pl.ds(gbase, _PER_W)], idx_v)

    def one_off(t):
        sl = pl.ds(t * _L, _L)
        lanes = lax.iota(jnp.int32, _L)
        p0 = gbase + t * _L
        bt = (p0 >> 10) & 7
        bc0 = p0 & 127
        v = idx_v[sl]
        idx_v[sl] = (
            ((v >> 3) << 13)
            + ((v & 7) << 7)
            + ((bt << 10) + bc0)
            + lanes
        )

    _NG = 2
    _GE = _PER_W // _NG
    _GV = _GE // _L

    def off_group(vbase, carry):
        def off_body(u, c):
            for j in range(4):
                one_off(vbase + u * 4 + j)
            return c
        return lax.fori_loop(0, _GV // 4, off_body, carry)

    def group_body(g, carry):
        carry = off_group(g * _GV, carry)
        sl = pl.ds(g * _GE, _GE)
        pltpu.make_async_copy(
            x1d_hbm.at[idx_v.at[sl]], out_v.at[sl], sem
        ).start()
        return carry

    lax.fori_loop(0, _NG, group_body, 0)

    def drain_body(g, carry):
        sl = pl.ds(g * _GE, _GE)
        pltpu.make_async_copy(
            x1d_hbm.at[idx_v.at[sl]], out_v.at[sl], sem
        ).wait()
        return carry

    lax.fori_loop(0, _NG, drain_body, 0)

    pltpu.sync_copy(out_v, out_hbm.at[pl.ds(gbase, _PER_W)])


@functools.partial(
    pl.kernel,
    out_type=jax.ShapeDtypeStruct((_TOTAL,), jnp.float32),
    mesh=plsc.VectorSubcoreMesh(core_axis_name="c", subcore_axis_name="s"),
    compiler_params=pltpu.CompilerParams(needs_layout_passes=False),
    scratch_types=[
        pltpu.VMEM((_PER_W,), jnp.int32),
        pltpu.VMEM((_PER_W,), jnp.float32),
        pltpu.SemaphoreType.DMA,
    ],
)
def _sc_gather(x1d_hbm, idx_hbm, out_hbm, idx_v, out_v, sem):
    _gather_body(x1d_hbm, idx_hbm, out_hbm, idx_v, out_v, sem)


def kernel(x, dim, index, sparse_grad):
    del dim, sparse_grad
    x1d = x.T.reshape(12500, 8, 8, 128).transpose(0, 2, 1, 3).reshape(_XN)
    idx1d = (
        index.T.reshape(25, 8, 8, 128).transpose(0, 2, 1, 3).reshape(_TOTAL)
    ).astype(jnp.int32)
    out1d = _sc_gather(x1d, idx1d)
    return out1d.reshape(25, 8, 8, 128).transpose(0, 2, 1, 3).reshape(_H, _B).T


# speedup vs baseline: 1.0070x; 1.0070x over previous
import functools

import jax
import jax.numpy as jnp
from jax import lax
from jax.experimental import pallas as pl
from jax.experimental.pallas import tpu as pltpu
from jax.experimental.pallas import tpu_sc as plsc

_B = 1024
_V = 100000
_H = 200
_L = 16

_NC = 2
_NS = 16
_NW = _NC * _NS
_TOTAL = _B * _H
_XN = _B * _V
_PER_W = _TOTAL // _NW
_CHUNK = 128
_STREAMS = _PER_W // _CHUNK
_FIRE = 10


def _gather_body(x1d_hbm, idx_hbm, out_hbm, idx_v, out_v, sem):
    wid = lax.axis_index("s") * _NC + lax.axis_index("c")
    gbase = wid * _PER_W
    pltpu.sync_copy(idx_hbm.at[pl.ds(gbase, _PER_W)], idx_v)

    def one_off(t):
        sl = pl.ds(t * _L, _L)
        lanes = lax.iota(jnp.int32, _L)
        p0 = gbase + t * _L
        bt = (p0 >> 10) & 7
        bc0 = p0 & 127
        v = idx_v[sl]
        idx_v[sl] = (
            ((v >> 3) << 13)
            + ((v & 7) << 7)
            + ((bt << 10) + bc0)
            + lanes
        )

    _NG = 4
    _GE = _PER_W // _NG
    _GV = _GE // _L

    def off_group(vbase, carry):
        def off_body(u, c):
            for j in range(4):
                one_off(vbase + u * 4 + j)
            return c
        return lax.fori_loop(0, _GV // 4, off_body, carry)

    def group_body(g, carry):
        carry = off_group(g * _GV, carry)
        sl = pl.ds(g * _GE, _GE)
        pltpu.make_async_copy(
            x1d_hbm.at[idx_v.at[sl]], out_v.at[sl], sem
        ).start()
        return carry

    lax.fori_loop(0, _NG, group_body, 0)

    def drain_body(g, carry):
        sl = pl.ds(g * _GE, _GE)
        pltpu.make_async_copy(
            x1d_hbm.at[idx_v.at[sl]], out_v.at[sl], sem
        ).wait()
        return carry

    lax.fori_loop(0, _NG, drain_body, 0)

    pltpu.sync_copy(out_v, out_hbm.at[pl.ds(gbase, _PER_W)])


@functools.partial(
    pl.kernel,
    out_type=jax.ShapeDtypeStruct((_TOTAL,), jnp.float32),
    mesh=plsc.VectorSubcoreMesh(core_axis_name="c", subcore_axis_name="s"),
    compiler_params=pltpu.CompilerParams(needs_layout_passes=False),
    scratch_types=[
        pltpu.VMEM((_PER_W,), jnp.int32),
        pltpu.VMEM((_PER_W,), jnp.float32),
        pltpu.SemaphoreType.DMA,
    ],
)
def _sc_gather(x1d_hbm, idx_hbm, out_hbm, idx_v, out_v, sem):
    _gather_body(x1d_hbm, idx_hbm, out_hbm, idx_v, out_v, sem)


def kernel(x, dim, index, sparse_grad):
    del dim, sparse_grad
    x1d = x.T.reshape(12500, 8, 8, 128).transpose(0, 2, 1, 3).reshape(_XN)
    idx1d = (
        index.T.reshape(25, 8, 8, 128).transpose(0, 2, 1, 3).reshape(_TOTAL)
    ).astype(jnp.int32)
    out1d = _sc_gather(x1d, idx1d)
    return out1d.reshape(25, 8, 8, 128).transpose(0, 2, 1, 3).reshape(_H, _B).T


# speedup vs baseline: 1.0089x; 1.0018x over previous
import functools

import jax
import jax.numpy as jnp
from jax import lax
from jax.experimental import pallas as pl
from jax.experimental.pallas import tpu as pltpu
from jax.experimental.pallas import tpu_sc as plsc

_B = 1024
_V = 100000
_H = 200
_L = 16

_NC = 2
_NS = 16
_NW = _NC * _NS
_TOTAL = _B * _H
_XN = _B * _V
_PER_W = _TOTAL // _NW
_CHUNK = 128
_STREAMS = _PER_W // _CHUNK
_FIRE = 10


def _gather_body(x1d_hbm, idx_hbm, out_hbm, idx_v, out_v, sem):
    wid = lax.axis_index("s") * _NC + lax.axis_index("c")
    gbase = wid * _PER_W
    pltpu.sync_copy(idx_hbm.at[pl.ds(gbase, _PER_W)], idx_v)

    def one_off(t):
        sl = pl.ds(t * _L, _L)
        lanes = lax.iota(jnp.int32, _L)
        p0 = gbase + t * _L
        bt = (p0 >> 10) & 7
        bc0 = p0 & 127
        v = idx_v[sl]
        idx_v[sl] = (
            ((v >> 3) << 13)
            + ((v & 7) << 7)
            + ((bt << 10) + bc0)
            + lanes
        )

    _NG = 8
    _GE = _PER_W // _NG
    _GV = _GE // _L

    def off_group(vbase, carry):
        def off_body(u, c):
            for j in range(4):
                one_off(vbase + u * 4 + j)
            return c
        return lax.fori_loop(0, _GV // 4, off_body, carry)

    def group_body(g, carry):
        carry = off_group(g * _GV, carry)
        sl = pl.ds(g * _GE, _GE)
        pltpu.make_async_copy(
            x1d_hbm.at[idx_v.at[sl]], out_v.at[sl], sem
        ).start()
        return carry

    lax.fori_loop(0, _NG, group_body, 0)

    def drain_body(g, carry):
        sl = pl.ds(g * _GE, _GE)
        pltpu.make_async_copy(
            x1d_hbm.at[idx_v.at[sl]], out_v.at[sl], sem
        ).wait()
        return carry

    lax.fori_loop(0, _NG, drain_body, 0)

    pltpu.sync_copy(out_v, out_hbm.at[pl.ds(gbase, _PER_W)])


@functools.partial(
    pl.kernel,
    out_type=jax.ShapeDtypeStruct((_TOTAL,), jnp.float32),
    mesh=plsc.VectorSubcoreMesh(core_axis_name="c", subcore_axis_name="s"),
    compiler_params=pltpu.CompilerParams(needs_layout_passes=False),
    scratch_types=[
        pltpu.VMEM((_PER_W,), jnp.int32),
        pltpu.VMEM((_PER_W,), jnp.float32),
        pltpu.SemaphoreType.DMA,
    ],
)
def _sc_gather(x1d_hbm, idx_hbm, out_hbm, idx_v, out_v, sem):
    _gather_body(x1d_hbm, idx_hbm, out_hbm, idx_v, out_v, sem)


def kernel(x, dim, index, sparse_grad):
    del dim, sparse_grad
    x1d = x.T.reshape(12500, 8, 8, 128).transpose(0, 2, 1, 3).reshape(_XN)
    idx1d = (
        index.T.reshape(25, 8, 8, 128).transpose(0, 2, 1, 3).reshape(_TOTAL)
    ).astype(jnp.int32)
    out1d = _sc_gather(x1d, idx1d)
    return out1d.reshape(25, 8, 8, 128).transpose(0, 2, 1, 3).reshape(_H, _B).T
